# Initial kernel scaffold; baseline (speedup 1.0000x reference)
import functools

import jax
import jax.numpy as jnp
from jax import lax
from jax.experimental import pallas as pl
from jax.experimental.pallas import tpu as pltpu
from jax.experimental.pallas import tpu_sc as plsc

DIM = 64
NC = 2
NS = 16
NW = NC * NS
IPS = 128
K = 4
RPI = IPS * K


@functools.cache
def _make_gather(V, D, B, interpret=False):
    assert B % (NW * RPI) == 0
    b_per_w = B // NW
    n_iter = b_per_w // RPI
    rows_per_w = b_per_w // IPS
    mesh = plsc.VectorSubcoreMesh(core_axis_name="c", subcore_axis_name="s")

    @functools.partial(
        pl.kernel,
        out_type=jax.ShapeDtypeStruct((B, D), jnp.float32),
        mesh=mesh,
        scratch_types=[
            pltpu.VMEM((K, IPS), jnp.int32),
            pltpu.VMEM((RPI, D), jnp.float32),
            pltpu.SemaphoreType.DMA,
        ],
        interpret=interpret,
    )
    def gather_kernel(table_hbm, idx_hbm, out_hbm, idx_v, rows_v, sem):
        wid = lax.axis_index("s") * NC + lax.axis_index("c")
        base_row = wid * rows_per_w

        @pl.loop(0, n_iter)
        def _(i):
            row_off = pl.multiple_of(base_row + i * K, K)
            flat_off = pl.multiple_of((base_row + i * K) * IPS, RPI)
            pltpu.sync_copy(idx_hbm.at[pl.ds(row_off, K)], idx_v)
            copies = []
            for j in range(K):
                copies.append(pltpu.async_copy(
                    table_hbm.at[idx_v.at[j]],
                    rows_v.at[pl.ds(j * IPS, IPS)],
                    sem,
                ))
            for c in copies:
                c.wait()
            pltpu.sync_copy(rows_v, out_hbm.at[pl.ds(flat_off, RPI)])

    return gather_kernel


def kernel(X, emb):
    batch, seq = X.shape
    B = batch * seq
    idx2d = X.reshape(-1, IPS).astype(jnp.int32)
    out = _make_gather(emb.shape[0], DIM, B)(emb, idx2d)
    return out.reshape(batch, seq, DIM)


# speedup vs baseline: 1.7975x; 1.7975x over previous
import functools

import jax
import jax.numpy as jnp
from jax import lax
from jax.experimental import pallas as pl
from jax.experimental.pallas import tpu as pltpu
from jax.experimental.pallas import tpu_sc as plsc

DIM = 64
NC = 2
NS = 16
NW = NC * NS
IPS = 128
K = 4
RPI = IPS * K


@functools.cache
def _make_gather(V, D, B, interpret=False):
    assert B % (NW * RPI) == 0
    b_per_w = B // NW
    n_iter = b_per_w // RPI
    rows_per_w = b_per_w // IPS
    mesh = plsc.VectorSubcoreMesh(core_axis_name="c", subcore_axis_name="s")

    @functools.partial(
        pl.kernel,
        out_type=jax.ShapeDtypeStruct((B, D), jnp.float32),
        mesh=mesh,
        scratch_types=[
            pltpu.VMEM((K, IPS), jnp.int32),
            pltpu.VMEM((RPI, D), jnp.float32),
            pltpu.SemaphoreType.DMA,
        ],
        compiler_params=pltpu.CompilerParams(use_tc_tiling_on_sc=False),
        interpret=interpret,
    )
    def gather_kernel(table_hbm, idx_hbm, out_hbm, idx_v, rows_v, sem):
        wid = lax.axis_index("s") * NC + lax.axis_index("c")
        base_row = wid * rows_per_w

        @pl.loop(0, n_iter)
        def _(i):
            row_off = pl.multiple_of(base_row + i * K, K)
            flat_off = pl.multiple_of((base_row + i * K) * IPS, RPI)
            pltpu.sync_copy(idx_hbm.at[pl.ds(row_off, K)], idx_v)
            copies = []
            for j in range(K):
                copies.append(pltpu.async_copy(
                    table_hbm.at[idx_v.at[j]],
                    rows_v.at[pl.ds(j * IPS, IPS)],
                    sem,
                ))
            for c in copies:
                c.wait()
            pltpu.sync_copy(rows_v, out_hbm.at[pl.ds(flat_off, RPI)])

    return gather_kernel


def kernel(X, emb):
    batch, seq = X.shape
    B = batch * seq
    idx2d = X.reshape(-1, IPS).astype(jnp.int32)
    out = _make_gather(emb.shape[0], DIM, B)(emb, idx2d)
    return out.reshape(batch, seq, DIM)


# speedup vs baseline: 1.8774x; 1.0444x over previous
import functools

import jax
import jax.numpy as jnp
from jax import lax
from jax.experimental import pallas as pl
from jax.experimental.pallas import tpu as pltpu
from jax.experimental.pallas import tpu_sc as plsc

DIM = 64
NC = 2
NS = 16
NW = NC * NS
IPS = 128
K = 4
RPI = IPS * K


@functools.cache
def _make_gather(V, D, B):
    assert B % (NW * RPI) == 0
    b_per_w = B // NW
    n_iter = b_per_w // RPI
    assert n_iter % 2 == 0
    rows_per_w = b_per_w // IPS
    mesh = plsc.VectorSubcoreMesh(core_axis_name="c", subcore_axis_name="s")

    @functools.partial(
        pl.kernel,
        out_type=jax.ShapeDtypeStruct((B, D), jnp.float32),
        mesh=mesh,
        scratch_types=[
            pltpu.VMEM((rows_per_w, IPS), jnp.int32),
            pltpu.VMEM((2, RPI, D), jnp.float32),
            pltpu.SemaphoreType.DMA,
            pltpu.SemaphoreType.DMA,
            pltpu.SemaphoreType.DMA,
        ],
        compiler_params=pltpu.CompilerParams(use_tc_tiling_on_sc=False),
    )
    def gather_kernel(table_hbm, idx_hbm, out_hbm, idx_v, rows_v, sem_g0,
                      sem_g1, sem_s):
        wid = lax.axis_index("s") * NC + lax.axis_index("c")
        base_row = wid * rows_per_w
        base_flat = wid * b_per_w
        sem_g = (sem_g0, sem_g1)

        pltpu.sync_copy(idx_hbm.at[pl.ds(base_row, rows_per_w)], idx_v)

        def fire(chunk, buf):
            for j in range(K):
                pltpu.async_copy(
                    table_hbm.at[idx_v.at[chunk * K + j]],
                    rows_v.at[buf].at[pl.ds(j * IPS, IPS)],
                    sem_g[buf],
                )

        def drain(chunk, buf):
            for j in range(K):
                pltpu.make_async_copy(
                    table_hbm.at[idx_v.at[chunk * K + j]],
                    rows_v.at[buf].at[pl.ds(j * IPS, IPS)],
                    sem_g[buf],
                ).wait()

        def store(chunk, buf):
            off = pl.multiple_of(base_flat + chunk * RPI, RPI)
            return pltpu.async_copy(
                rows_v.at[buf], out_hbm.at[pl.ds(off, RPI)], sem_s)

        def store_wait(chunk, buf):
            off = pl.multiple_of(base_flat + chunk * RPI, RPI)
            pltpu.make_async_copy(
                rows_v.at[buf], out_hbm.at[pl.ds(off, RPI)], sem_s).wait()

        fire(0, 0)

        @pl.loop(0, n_iter, step=2)
        def _(i0):
            for b in range(2):
                i = i0 + b
                ob = 1 - b

                @pl.when(i > 0)
                def _():
                    store_wait(i - 1, ob)

                @pl.when(i + 1 < n_iter)
                def _():
                    fire(i + 1, ob)

                drain(i, b)
                store(i, b)

        store_wait(n_iter - 1, (n_iter - 1) % 2)

    return gather_kernel


def kernel(X, emb):
    batch, seq = X.shape
    B = batch * seq
    idx2d = X.reshape(-1, IPS).astype(jnp.int32)
    out = _make_gather(emb.shape[0], DIM, B)(emb, idx2d)
    return out.reshape(batch, seq, DIM)
